# Initial kernel scaffold; baseline (speedup 1.0000x reference)
import jax
import jax.numpy as jnp
from jax.experimental import pallas as pl


def kernel(user_feat, item_feat, rating_feat, row_idxs, col_idxs, rating, gu_w1, gu_b1, gu_w2, gu_b2, att_w1, att_b1, att_w2, att_b2, att_w3, att_b3, w_w, w_b):
    raise NotImplementedError("write your pallas kernel here")


# speedup vs baseline: 5.4712x; 5.4712x over previous
import functools

import jax
import jax.numpy as jnp
from jax import lax
from jax.experimental import pallas as pl
from jax.experimental.pallas import tpu as pltpu
from jax.experimental.pallas import tpu_sc as plsc

U_N, I_N, E_N, D_N, R_N = 8000, 2000, 320000, 128, 5

NC, NS = 2, 16
NW = NC * NS
EPW = E_N // NW

KC = 80
GA = 5
GE = KC * GA
NGRP = EPW // GE

KS = 80
NCH = EPW // KS
IPAD = 2048
IPT = IPAD // NS

BE = 512
NB = E_N // BE

@functools.lru_cache(maxsize=None)
def _build_gather_stage():
    mesh = plsc.VectorSubcoreMesh(
        core_axis_name="c", subcore_axis_name="s",
        num_cores=NC, num_subcores=NS)
    return functools.partial(
        pl.kernel,
        out_type=[
            jax.ShapeDtypeStruct((E_N, D_N), jnp.float32),
            jax.ShapeDtypeStruct((E_N, D_N), jnp.float32),
        ],
        mesh=mesh,
        scratch_types=[
            pltpu.VMEM((GE,), jnp.int32),
            pltpu.VMEM((GE,), jnp.int32),
            pltpu.VMEM((GE, D_N), jnp.float32),
            pltpu.VMEM((GE, D_N), jnp.float32),
            pltpu.SemaphoreType.DMA,
        ],
    )(_gather_body)


def _gather_body(uf, itf, rowi, coli, ug, ig, ridx, cidx, ubuf, ibuf, sem):
    cid = lax.axis_index("c")
    sid = lax.axis_index("s")
    wid = sid * NC + cid
    base = wid * EPW

    def grp(g, carry):
        off = base + g * GE
        pltpu.sync_copy(rowi.at[pl.ds(off, GE)], ridx)
        pltpu.sync_copy(coli.at[pl.ds(off, GE)], cidx)
        cps = []
        for j in range(GA):
            cps.append(pltpu.async_copy(
                uf.at[ridx.at[pl.ds(j * KC, KC)]],
                ubuf.at[pl.ds(j * KC, KC)], sem))
            cps.append(pltpu.async_copy(
                itf.at[cidx.at[pl.ds(j * KC, KC)]],
                ibuf.at[pl.ds(j * KC, KC)], sem))
        for c in cps:
            c.wait()
        pltpu.sync_copy(ubuf, ug.at[pl.ds(off, GE)])
        pltpu.sync_copy(ibuf, ig.at[pl.ds(off, GE)])
        return carry

    lax.fori_loop(0, NGRP, grp, 0)


def _mlp_body(rt_ref, u_ref, i_ref, w1u_ref, rf8_ref, w1r_ref, b1_ref,
              w2_ref, b2_ref, a1f_ref, a1i_ref, ab1_ref, a2_ref, ab2_ref,
              a3_ref, ab3_ref, m_ref, ew_ref):
    u = u_ref[...]
    iv = i_ref[...]
    r = rt_ref[0]
    oh = (r == lax.broadcasted_iota(jnp.int32, (BE, 8), 1)).astype(jnp.float32)
    rp = jnp.dot(rf8_ref[...], w1r_ref[...], preferred_element_type=jnp.float32)
    h = jnp.maximum(
        jnp.dot(u, w1u_ref[...], preferred_element_type=jnp.float32)
        + jnp.dot(oh, rp, preferred_element_type=jnp.float32)
        + b1_ref[...], 0.0)
    f = jnp.maximum(
        jnp.dot(h, w2_ref[...], preferred_element_type=jnp.float32)
        + b2_ref[...], 0.0)
    a = jnp.maximum(
        jnp.dot(f, a1f_ref[...], preferred_element_type=jnp.float32)
        + jnp.dot(iv, a1i_ref[...], preferred_element_type=jnp.float32)
        + ab1_ref[...], 0.0)
    a = jnp.maximum(
        jnp.dot(a, a2_ref[...], preferred_element_type=jnp.float32)
        + ab2_ref[...], 0.0)
    w8 = jnp.dot(a, a3_ref[...], preferred_element_type=jnp.float32) + ab3_ref[...]
    ew = jnp.exp(w8[:, 0:1])
    m_ref[...] = f * ew
    ew_ref[...] = jnp.broadcast_to(ew, (BE, D_N))


@functools.lru_cache(maxsize=None)
def _build_scatter_stage():
    mesh = plsc.VectorSubcoreMesh(
        core_axis_name="c", subcore_axis_name="s",
        num_cores=NC, num_subcores=NS)
    return functools.partial(
        pl.kernel,
        out_type=[
            jax.ShapeDtypeStruct((NC, IPAD, D_N), jnp.float32),
            jax.ShapeDtypeStruct((NC, IPAD, D_N), jnp.float32),
        ],
        mesh=mesh,
        scratch_types=[
            pltpu.VMEM((1, KS), jnp.int32),
            pltpu.VMEM((KS, D_N), jnp.float32),
            pltpu.VMEM((KS, D_N), jnp.float32),
            pltpu.VMEM((IPT, D_N), jnp.float32),
            pltpu.VMEM_SHARED((IPAD, D_N), jnp.float32),
            pltpu.VMEM_SHARED((IPAD, D_N), jnp.float32),
        ],
    )(_scatter_body)


def _scatter_body(m, ew, coli, zn, np_out, dp_out,
                  idxb, mbuf, ebuf, nbuf, accN, accD):
    cid = lax.axis_index("c")
    sid = lax.axis_index("s")
    wid = sid * NC + cid
    base = wid * EPW
    pltpu.sync_copy(zn.at[pl.ds(sid * IPT, IPT)], nbuf)
    pltpu.sync_copy(nbuf, accN.at[pl.ds(sid * IPT, IPT)])
    pltpu.sync_copy(nbuf, accD.at[pl.ds(sid * IPT, IPT)])
    plsc.subcore_barrier()

    def chunk(c, carry):
        off = base + c * KS
        pltpu.sync_copy(coli.at[pl.ds(off, KS)], idxb.at[0])
        pltpu.sync_copy(m.at[pl.ds(off, KS)], mbuf)
        pltpu.sync_copy(ew.at[pl.ds(off, KS)], ebuf)
        pltpu.sync_copy(mbuf, accN.at[idxb.at[0]], add=True)
        pltpu.sync_copy(ebuf, accD.at[idxb.at[0]], add=True)
        return carry

    lax.fori_loop(0, NCH, chunk, 0)
    plsc.subcore_barrier()
    pltpu.sync_copy(accN.at[pl.ds(sid * IPT, IPT)], nbuf)
    pltpu.sync_copy(nbuf, np_out.at[cid].at[pl.ds(sid * IPT, IPT)])
    pltpu.sync_copy(accD.at[pl.ds(sid * IPT, IPT)], nbuf)
    pltpu.sync_copy(nbuf, dp_out.at[cid].at[pl.ds(sid * IPT, IPT)])


def _final_body(np_ref, dp_ref, ww_ref, wb_ref, z_ref):
    n = np_ref[0] + np_ref[1]
    d = dp_ref[0, :, 0:1] + dp_ref[1, :, 0:1]
    d = jnp.where(d > 0, d, 1.0)
    z_ref[...] = (
        jnp.dot(n / d, ww_ref[...], preferred_element_type=jnp.float32)
        + wb_ref[...])


def kernel(user_feat, item_feat, rating_feat, row_idxs, col_idxs, rating,
           gu_w1, gu_b1, gu_w2, gu_b2,
           att_w1, att_b1, att_w2, att_b2, att_w3, att_b3,
           w_w, w_b):
    row_i = row_idxs.astype(jnp.int32)
    col_i = col_idxs.astype(jnp.int32)
    rat_i = rating.astype(jnp.int32)

    ug, ig = _build_gather_stage()(user_feat, item_feat, row_i, col_i)

    w1u = gu_w1[:, :D_N].T
    w1r = gu_w1[:, D_N:].T
    rf8 = jnp.zeros((8, D_N), jnp.float32).at[:R_N].set(rating_feat)
    w2t = gu_w2.T
    a1ft = att_w1[:, :D_N].T
    a1it = att_w1[:, D_N:].T
    a2t = att_w2.T
    a3p = jnp.zeros((D_N, 8), jnp.float32).at[:, 0].set(att_w3[0])
    b1_2 = gu_b1.reshape(1, D_N)
    b2_2 = gu_b2.reshape(1, D_N)
    ab1_2 = att_b1.reshape(1, D_N)
    ab2_2 = att_b2.reshape(1, D_N)
    ab3_2 = jnp.broadcast_to(att_b3.reshape(1, 1), (1, 8))
    rt3 = rat_i.reshape(NB, BE, 1)

    full = lambda shp: pl.BlockSpec(shp, lambda b: tuple(0 for _ in shp))
    m, ew16 = pl.pallas_call(
        _mlp_body,
        grid=(NB,),
        in_specs=[
            pl.BlockSpec((1, BE, 1), lambda b: (b, 0, 0)),
            pl.BlockSpec((BE, D_N), lambda b: (b, 0)),
            pl.BlockSpec((BE, D_N), lambda b: (b, 0)),
            full((D_N, D_N)),
            full((8, D_N)),
            full((D_N, D_N)),
            full((1, D_N)),
            full((D_N, D_N)),
            full((1, D_N)),
            full((D_N, D_N)),
            full((D_N, D_N)),
            full((1, D_N)),
            full((D_N, D_N)),
            full((1, D_N)),
            full((D_N, 8)),
            full((1, 8)),
        ],
        out_specs=[
            pl.BlockSpec((BE, D_N), lambda b: (b, 0)),
            pl.BlockSpec((BE, D_N), lambda b: (b, 0)),
        ],
        out_shape=[
            jax.ShapeDtypeStruct((E_N, D_N), jnp.float32),
            jax.ShapeDtypeStruct((E_N, D_N), jnp.float32),
        ],
    )(rt3, ug, ig, w1u, rf8, w1r, b1_2, w2t, b2_2,
      a1ft, a1it, ab1_2, a2t, ab2_2, a3p, ab3_2)

    zn = jnp.zeros((IPAD, D_N), jnp.float32)
    np_part, dp_part = _build_scatter_stage()(m, ew16, col_i, zn)

    z = pl.pallas_call(
        _final_body,
        in_specs=[
            pl.BlockSpec((NC, IPAD, D_N), lambda: (0, 0, 0)),
            pl.BlockSpec((NC, IPAD, D_N), lambda: (0, 0, 0)),
            pl.BlockSpec((D_N, D_N), lambda: (0, 0)),
            pl.BlockSpec((1, D_N), lambda: (0, 0)),
        ],
        out_specs=pl.BlockSpec((IPAD, D_N), lambda: (0, 0)),
        out_shape=jax.ShapeDtypeStruct((IPAD, D_N), jnp.float32),
    )(np_part, dp_part, w_w.T, w_b.reshape(1, D_N))
    return z[:I_N]


# speedup vs baseline: 6.3422x; 1.1592x over previous
import functools

import jax
import jax.numpy as jnp
from jax import lax
from jax.experimental import pallas as pl
from jax.experimental.pallas import tpu as pltpu
from jax.experimental.pallas import tpu_sc as plsc

U_N, I_N, E_N, D_N, R_N = 8000, 2000, 320000, 128, 5

NC, NS = 2, 16
NW = NC * NS
EPW = E_N // NW

ETW = E_N // NS
KC = 80
GA = 5
GE = KC * GA
NGRP = ETW // GE

KS = 80
NCH = EPW // KS
IPAD = 2048
IPT = IPAD // NS

BE = 512
NB = E_N // BE

@functools.lru_cache(maxsize=None)
def _build_gather_stage():
    mesh = plsc.VectorSubcoreMesh(
        core_axis_name="c", subcore_axis_name="s",
        num_cores=NC, num_subcores=NS)
    return functools.partial(
        pl.kernel,
        out_type=[
            jax.ShapeDtypeStruct((E_N, D_N), jnp.float32),
            jax.ShapeDtypeStruct((E_N, D_N), jnp.float32),
        ],
        mesh=mesh,
        scratch_types=[
            pltpu.VMEM((2, GA, KC), jnp.int32),
            pltpu.VMEM((2, GE, D_N), jnp.float32),
            pltpu.SemaphoreType.DMA,
            pltpu.SemaphoreType.DMA,
        ],
    )(_gather_body)


def _gather_body(uf, itf, rowi, coli, ug, ig, idx, buf, gsem, wsem):
    cid = lax.axis_index("c")
    sid = lax.axis_index("s")
    base = sid * ETW

    def run(table, idx_src, out):
        def grp(g, carry):
            b = lax.rem(g, 2)
            off = base + g * GE

            @pl.when(g >= 2)
            def _drain_write():
                pltpu.make_async_copy(
                    buf.at[b], out.at[pl.ds(off, GE)], wsem).wait()

            for j in range(GA):
                pltpu.sync_copy(idx_src.at[pl.ds(off + j * KC, KC)],
                                idx.at[b, j])
            cps = [
                pltpu.async_copy(
                    table.at[idx.at[b, j]],
                    buf.at[b].at[pl.ds(j * KC, KC)], gsem)
                for j in range(GA)
            ]
            for c in cps:
                c.wait()
            pltpu.async_copy(buf.at[b], out.at[pl.ds(off, GE)], wsem)
            return carry

        lax.fori_loop(0, NGRP, grp, 0)
        for b in range(2):
            pltpu.make_async_copy(
                buf.at[b], out.at[pl.ds(base, GE)], wsem).wait()

    @pl.when(cid == 0)
    def _user():
        run(uf, rowi, ug)

    @pl.when(cid == 1)
    def _item():
        run(itf, coli, ig)


def _mlp_body(rt_ref, u_ref, i_ref, w1u_ref, rf8_ref, w1r_ref, b1_ref,
              w2_ref, b2_ref, a1f_ref, a1i_ref, ab1_ref, a2_ref, ab2_ref,
              a3_ref, ab3_ref, m_ref, ew_ref):
    u = u_ref[...]
    iv = i_ref[...]
    r = rt_ref[0]
    oh = (r == lax.broadcasted_iota(jnp.int32, (BE, 8), 1)).astype(jnp.float32)
    rp = jnp.dot(rf8_ref[...], w1r_ref[...], preferred_element_type=jnp.float32)
    h = jnp.maximum(
        jnp.dot(u, w1u_ref[...], preferred_element_type=jnp.float32)
        + jnp.dot(oh, rp, preferred_element_type=jnp.float32)
        + b1_ref[...], 0.0)
    f = jnp.maximum(
        jnp.dot(h, w2_ref[...], preferred_element_type=jnp.float32)
        + b2_ref[...], 0.0)
    a = jnp.maximum(
        jnp.dot(f, a1f_ref[...], preferred_element_type=jnp.float32)
        + jnp.dot(iv, a1i_ref[...], preferred_element_type=jnp.float32)
        + ab1_ref[...], 0.0)
    a = jnp.maximum(
        jnp.dot(a, a2_ref[...], preferred_element_type=jnp.float32)
        + ab2_ref[...], 0.0)
    w8 = jnp.dot(a, a3_ref[...], preferred_element_type=jnp.float32) + ab3_ref[...]
    ew = jnp.exp(w8[:, 0:1])
    m_ref[...] = f * ew
    ew_ref[...] = jnp.broadcast_to(ew, (BE, D_N))


@functools.lru_cache(maxsize=None)
def _build_scatter_stage():
    mesh = plsc.VectorSubcoreMesh(
        core_axis_name="c", subcore_axis_name="s",
        num_cores=NC, num_subcores=NS)
    return functools.partial(
        pl.kernel,
        out_type=[
            jax.ShapeDtypeStruct((NC, IPAD, D_N), jnp.float32),
            jax.ShapeDtypeStruct((NC, IPAD, D_N), jnp.float32),
        ],
        mesh=mesh,
        scratch_types=[
            pltpu.VMEM((2, KS), jnp.int32),
            pltpu.VMEM((2, KS, D_N), jnp.float32),
            pltpu.VMEM((2, KS, D_N), jnp.float32),
            pltpu.VMEM((IPT, D_N), jnp.float32),
            pltpu.SemaphoreType.DMA,
            pltpu.VMEM_SHARED((IPAD, D_N), jnp.float32),
            pltpu.VMEM_SHARED((IPAD, D_N), jnp.float32),
        ],
    )(_scatter_body)


def _scatter_body(m, ew, coli, zn, np_out, dp_out,
                  idxb, mbuf, ebuf, nbuf, lsem, accN, accD):
    cid = lax.axis_index("c")
    sid = lax.axis_index("s")
    wid = sid * NC + cid
    base = wid * EPW

    def fire_loads(c, b):
        off = base + c * KS
        pltpu.async_copy(coli.at[pl.ds(off, KS)], idxb.at[b], lsem)
        pltpu.async_copy(m.at[pl.ds(off, KS)], mbuf.at[b], lsem)
        pltpu.async_copy(ew.at[pl.ds(off, KS)], ebuf.at[b], lsem)

    fire_loads(0, 0)
    pltpu.sync_copy(zn.at[pl.ds(sid * IPT, IPT)], nbuf)
    pltpu.sync_copy(nbuf, accN.at[pl.ds(sid * IPT, IPT)])
    pltpu.sync_copy(nbuf, accD.at[pl.ds(sid * IPT, IPT)])
    plsc.subcore_barrier()

    def chunk(c, carry):
        b = lax.rem(c, 2)
        off = base + c * KS
        pltpu.make_async_copy(coli.at[pl.ds(off, KS)], idxb.at[b], lsem).wait()
        pltpu.make_async_copy(m.at[pl.ds(off, KS)], mbuf.at[b], lsem).wait()
        pltpu.make_async_copy(ew.at[pl.ds(off, KS)], ebuf.at[b], lsem).wait()

        @pl.when(c + 1 < NCH)
        def _prefetch():
            fire_loads(c + 1, 1 - b)

        pltpu.sync_copy(mbuf.at[b], accN.at[idxb.at[b]], add=True)
        pltpu.sync_copy(ebuf.at[b], accD.at[idxb.at[b]], add=True)
        return carry

    lax.fori_loop(0, NCH, chunk, 0)
    plsc.subcore_barrier()
    pltpu.sync_copy(accN.at[pl.ds(sid * IPT, IPT)], nbuf)
    pltpu.sync_copy(nbuf, np_out.at[cid].at[pl.ds(sid * IPT, IPT)])
    pltpu.sync_copy(accD.at[pl.ds(sid * IPT, IPT)], nbuf)
    pltpu.sync_copy(nbuf, dp_out.at[cid].at[pl.ds(sid * IPT, IPT)])


def _final_body(np_ref, dp_ref, ww_ref, wb_ref, z_ref):
    n = np_ref[0] + np_ref[1]
    d = dp_ref[0, :, 0:1] + dp_ref[1, :, 0:1]
    d = jnp.where(d > 0, d, 1.0)
    z_ref[...] = (
        jnp.dot(n / d, ww_ref[...], preferred_element_type=jnp.float32)
        + wb_ref[...])


def kernel(user_feat, item_feat, rating_feat, row_idxs, col_idxs, rating,
           gu_w1, gu_b1, gu_w2, gu_b2,
           att_w1, att_b1, att_w2, att_b2, att_w3, att_b3,
           w_w, w_b):
    row_i = row_idxs.astype(jnp.int32)
    col_i = col_idxs.astype(jnp.int32)
    rat_i = rating.astype(jnp.int32)

    ug, ig = _build_gather_stage()(user_feat, item_feat, row_i, col_i)

    w1u = gu_w1[:, :D_N].T
    w1r = gu_w1[:, D_N:].T
    rf8 = jnp.zeros((8, D_N), jnp.float32).at[:R_N].set(rating_feat)
    w2t = gu_w2.T
    a1ft = att_w1[:, :D_N].T
    a1it = att_w1[:, D_N:].T
    a2t = att_w2.T
    a3p = jnp.zeros((D_N, 8), jnp.float32).at[:, 0].set(att_w3[0])
    b1_2 = gu_b1.reshape(1, D_N)
    b2_2 = gu_b2.reshape(1, D_N)
    ab1_2 = att_b1.reshape(1, D_N)
    ab2_2 = att_b2.reshape(1, D_N)
    ab3_2 = jnp.broadcast_to(att_b3.reshape(1, 1), (1, 8))
    rt3 = rat_i.reshape(NB, BE, 1)

    full = lambda shp: pl.BlockSpec(shp, lambda b: tuple(0 for _ in shp))
    m, ew16 = pl.pallas_call(
        _mlp_body,
        grid=(NB,),
        in_specs=[
            pl.BlockSpec((1, BE, 1), lambda b: (b, 0, 0)),
            pl.BlockSpec((BE, D_N), lambda b: (b, 0)),
            pl.BlockSpec((BE, D_N), lambda b: (b, 0)),
            full((D_N, D_N)),
            full((8, D_N)),
            full((D_N, D_N)),
            full((1, D_N)),
            full((D_N, D_N)),
            full((1, D_N)),
            full((D_N, D_N)),
            full((D_N, D_N)),
            full((1, D_N)),
            full((D_N, D_N)),
            full((1, D_N)),
            full((D_N, 8)),
            full((1, 8)),
        ],
        out_specs=[
            pl.BlockSpec((BE, D_N), lambda b: (b, 0)),
            pl.BlockSpec((BE, D_N), lambda b: (b, 0)),
        ],
        out_shape=[
            jax.ShapeDtypeStruct((E_N, D_N), jnp.float32),
            jax.ShapeDtypeStruct((E_N, D_N), jnp.float32),
        ],
    )(rt3, ug, ig, w1u, rf8, w1r, b1_2, w2t, b2_2,
      a1ft, a1it, ab1_2, a2t, ab2_2, a3p, ab3_2)

    zn = jnp.zeros((IPAD, D_N), jnp.float32)
    np_part, dp_part = _build_scatter_stage()(m, ew16, col_i, zn)

    z = pl.pallas_call(
        _final_body,
        in_specs=[
            pl.BlockSpec((NC, IPAD, D_N), lambda: (0, 0, 0)),
            pl.BlockSpec((NC, IPAD, D_N), lambda: (0, 0, 0)),
            pl.BlockSpec((D_N, D_N), lambda: (0, 0)),
            pl.BlockSpec((1, D_N), lambda: (0, 0)),
        ],
        out_specs=pl.BlockSpec((IPAD, D_N), lambda: (0, 0)),
        out_shape=jax.ShapeDtypeStruct((IPAD, D_N), jnp.float32),
    )(np_part, dp_part, w_w.T, w_b.reshape(1, D_N))
    return z[:I_N]


# speedup vs baseline: 7.8970x; 1.2452x over previous
import functools

import jax
import jax.numpy as jnp
from jax import lax
from jax.experimental import pallas as pl
from jax.experimental.pallas import tpu as pltpu
from jax.experimental.pallas import tpu_sc as plsc

U_N, I_N, E_N, D_N, R_N = 8000, 2000, 320000, 128, 5

NC, NS = 2, 16
NW = NC * NS

NSLAB = 2
ES = E_N // NSLAB

KC = 80
GA = 5
GE = KC * GA

IPAD = 2048
IPT = IPAD // NS

BE = 1000
NB = ES // BE


def _largest_chunk(n):
    for k in range(128, 0, -8):
        if n % k == 0:
            return k
    raise ValueError(n)


def _mesh():
    return plsc.VectorSubcoreMesh(
        core_axis_name="c", subcore_axis_name="s",
        num_cores=NC, num_subcores=NS)


@functools.lru_cache(maxsize=None)
def _build_gather_stage(es):
    etw = es // NS
    ngrp = etw // GE

    def _gather_body(uf, itf, rowi, coli, ug, ig, idx, buf, gsem, wsem):
        cid = lax.axis_index("c")
        sid = lax.axis_index("s")
        base = sid * etw

        def run(table, idx_src, out):
            def grp(g, carry):
                b = lax.rem(g, 2)
                off = base + g * GE

                @pl.when(g >= 2)
                def _drain_write():
                    pltpu.make_async_copy(
                        buf.at[b], out.at[pl.ds(off, GE)], wsem).wait()

                for j in range(GA):
                    pltpu.sync_copy(idx_src.at[pl.ds(off + j * KC, KC)],
                                    idx.at[b, j])
                cps = [
                    pltpu.async_copy(
                        table.at[idx.at[b, j]],
                        buf.at[b].at[pl.ds(j * KC, KC)], gsem)
                    for j in range(GA)
                ]
                for c in cps:
                    c.wait()
                pltpu.async_copy(buf.at[b], out.at[pl.ds(off, GE)], wsem)
                return carry

            lax.fori_loop(0, ngrp, grp, 0)
            for b in range(2):
                pltpu.make_async_copy(
                    buf.at[b], out.at[pl.ds(base, GE)], wsem).wait()

        @pl.when(cid == 0)
        def _user():
            run(uf, rowi, ug)

        @pl.when(cid == 1)
        def _item():
            run(itf, coli, ig)

    return functools.partial(
        pl.kernel,
        out_type=[
            jax.ShapeDtypeStruct((es, D_N), jnp.float32),
            jax.ShapeDtypeStruct((es, D_N), jnp.float32),
        ],
        mesh=_mesh(),
        scratch_types=[
            pltpu.VMEM((2, GA, KC), jnp.int32),
            pltpu.VMEM((2, GE, D_N), jnp.float32),
            pltpu.SemaphoreType.DMA,
            pltpu.SemaphoreType.DMA,
        ],
    )(_gather_body)


def _mlp_body(rt_ref, u_ref, i_ref, w1u_ref, rf8_ref, w1r_ref, b1_ref,
              w2_ref, b2_ref, a1f_ref, a1i_ref, ab1_ref, a2_ref, ab2_ref,
              a3_ref, ab3_ref, m_ref, ew_ref):
    u = u_ref[...]
    iv = i_ref[...]
    r = rt_ref[0]
    oh = (r == lax.broadcasted_iota(jnp.int32, (BE, 8), 1)).astype(jnp.float32)
    rp = jnp.dot(rf8_ref[...], w1r_ref[...], preferred_element_type=jnp.float32)
    h = jnp.maximum(
        jnp.dot(u, w1u_ref[...], preferred_element_type=jnp.float32)
        + jnp.dot(oh, rp, preferred_element_type=jnp.float32)
        + b1_ref[...], 0.0)
    f = jnp.maximum(
        jnp.dot(h, w2_ref[...], preferred_element_type=jnp.float32)
        + b2_ref[...], 0.0)
    a = jnp.maximum(
        jnp.dot(f, a1f_ref[...], preferred_element_type=jnp.float32)
        + jnp.dot(iv, a1i_ref[...], preferred_element_type=jnp.float32)
        + ab1_ref[...], 0.0)
    a = jnp.maximum(
        jnp.dot(a, a2_ref[...], preferred_element_type=jnp.float32)
        + ab2_ref[...], 0.0)
    w8 = jnp.dot(a, a3_ref[...], preferred_element_type=jnp.float32) + ab3_ref[...]
    ew = jnp.exp(w8[:, 0:1])
    m_ref[...] = f * ew
    ew_ref[...] = jnp.broadcast_to(ew, (BE, D_N))


@functools.lru_cache(maxsize=None)
def _build_scatter_stage(es):
    epw = es // NW
    ks = _largest_chunk(epw)
    nch = epw // ks

    def _scatter_body(m, ew, coli, n_init, d_init, np_out, dp_out,
                      idxb, mbuf, ebuf, nbuf, lsem, accN, accD):
        cid = lax.axis_index("c")
        sid = lax.axis_index("s")
        wid = sid * NC + cid
        base = wid * epw

        def fire_loads(c, b):
            off = base + c * ks
            pltpu.async_copy(coli.at[pl.ds(off, ks)], idxb.at[b], lsem)
            pltpu.async_copy(m.at[pl.ds(off, ks)], mbuf.at[b], lsem)
            pltpu.async_copy(ew.at[pl.ds(off, ks)], ebuf.at[b], lsem)

        fire_loads(0, 0)
        pltpu.sync_copy(n_init.at[cid].at[pl.ds(sid * IPT, IPT)], nbuf)
        pltpu.sync_copy(nbuf, accN.at[pl.ds(sid * IPT, IPT)])
        pltpu.sync_copy(d_init.at[cid].at[pl.ds(sid * IPT, IPT)], nbuf)
        pltpu.sync_copy(nbuf, accD.at[pl.ds(sid * IPT, IPT)])
        plsc.subcore_barrier()

        def chunk(c, carry):
            b = lax.rem(c, 2)
            off = base + c * ks
            pltpu.make_async_copy(
                coli.at[pl.ds(off, ks)], idxb.at[b], lsem).wait()
            pltpu.make_async_copy(
                m.at[pl.ds(off, ks)], mbuf.at[b], lsem).wait()
            pltpu.make_async_copy(
                ew.at[pl.ds(off, ks)], ebuf.at[b], lsem).wait()

            @pl.when(c + 1 < nch)
            def _prefetch():
                fire_loads(c + 1, 1 - b)

            pltpu.sync_copy(mbuf.at[b], accN.at[idxb.at[b]], add=True)
            pltpu.sync_copy(ebuf.at[b], accD.at[idxb.at[b]], add=True)
            return carry

        lax.fori_loop(0, nch, chunk, 0)
        plsc.subcore_barrier()
        pltpu.sync_copy(accN.at[pl.ds(sid * IPT, IPT)], nbuf)
        pltpu.sync_copy(nbuf, np_out.at[cid].at[pl.ds(sid * IPT, IPT)])
        pltpu.sync_copy(accD.at[pl.ds(sid * IPT, IPT)], nbuf)
        pltpu.sync_copy(nbuf, dp_out.at[cid].at[pl.ds(sid * IPT, IPT)])

    return functools.partial(
        pl.kernel,
        out_type=[
            jax.ShapeDtypeStruct((NC, IPAD, D_N), jnp.float32),
            jax.ShapeDtypeStruct((NC, IPAD, D_N), jnp.float32),
        ],
        mesh=_mesh(),
        scratch_types=[
            pltpu.VMEM((2, ks), jnp.int32),
            pltpu.VMEM((2, ks, D_N), jnp.float32),
            pltpu.VMEM((2, ks, D_N), jnp.float32),
            pltpu.VMEM((IPT, D_N), jnp.float32),
            pltpu.SemaphoreType.DMA,
            pltpu.VMEM_SHARED((IPAD, D_N), jnp.float32),
            pltpu.VMEM_SHARED((IPAD, D_N), jnp.float32),
        ],
    )(_scatter_body)


def _final_body(np_ref, dp_ref, ww_ref, wb_ref, z_ref):
    n = np_ref[0] + np_ref[1]
    d = dp_ref[0, :, 0:1] + dp_ref[1, :, 0:1]
    d = jnp.where(d > 0, d, 1.0)
    z_ref[...] = (
        jnp.dot(n / d, ww_ref[...], preferred_element_type=jnp.float32)
        + wb_ref[...])


def _mlp_call(rt3, ug, ig, weights):
    full = lambda shp: pl.BlockSpec(shp, lambda b: tuple(0 for _ in shp))
    return pl.pallas_call(
        _mlp_body,
        grid=(NB,),
        in_specs=[
            pl.BlockSpec((1, BE, 1), lambda b: (b, 0, 0)),
            pl.BlockSpec((BE, D_N), lambda b: (b, 0)),
            pl.BlockSpec((BE, D_N), lambda b: (b, 0)),
            full((D_N, D_N)),
            full((8, D_N)),
            full((D_N, D_N)),
            full((1, D_N)),
            full((D_N, D_N)),
            full((1, D_N)),
            full((D_N, D_N)),
            full((D_N, D_N)),
            full((1, D_N)),
            full((D_N, D_N)),
            full((1, D_N)),
            full((D_N, 8)),
            full((1, 8)),
        ],
        out_specs=[
            pl.BlockSpec((BE, D_N), lambda b: (b, 0)),
            pl.BlockSpec((BE, D_N), lambda b: (b, 0)),
        ],
        out_shape=[
            jax.ShapeDtypeStruct((ES, D_N), jnp.float32),
            jax.ShapeDtypeStruct((ES, D_N), jnp.float32),
        ],
    )(rt3, ug, ig, *weights)


def kernel(user_feat, item_feat, rating_feat, row_idxs, col_idxs, rating,
           gu_w1, gu_b1, gu_w2, gu_b2,
           att_w1, att_b1, att_w2, att_b2, att_w3, att_b3,
           w_w, w_b):
    row_i = row_idxs.astype(jnp.int32)
    col_i = col_idxs.astype(jnp.int32)
    rat_i = rating.astype(jnp.int32)

    w1u = gu_w1[:, :D_N].T
    w1r = gu_w1[:, D_N:].T
    rf8 = jnp.zeros((8, D_N), jnp.float32).at[:R_N].set(rating_feat)
    w2t = gu_w2.T
    a1ft = att_w1[:, :D_N].T
    a1it = att_w1[:, D_N:].T
    a2t = att_w2.T
    a3p = jnp.zeros((D_N, 8), jnp.float32).at[:, 0].set(att_w3[0])
    b1_2 = gu_b1.reshape(1, D_N)
    b2_2 = gu_b2.reshape(1, D_N)
    ab1_2 = att_b1.reshape(1, D_N)
    ab2_2 = att_b2.reshape(1, D_N)
    ab3_2 = jnp.broadcast_to(att_b3.reshape(1, 1), (1, 8))
    weights = (w1u, rf8, w1r, b1_2, w2t, b2_2,
               a1ft, a1it, ab1_2, a2t, ab2_2, a3p, ab3_2)

    gather = _build_gather_stage(ES)
    scatter = _build_scatter_stage(ES)

    np_part = jnp.zeros((NC, IPAD, D_N), jnp.float32)
    dp_part = jnp.zeros((NC, IPAD, D_N), jnp.float32)
    for s in range(NSLAB):
        sl = slice(s * ES, (s + 1) * ES)
        ug, ig = gather(user_feat, item_feat, row_i[sl], col_i[sl])
        rt3 = rat_i[sl].reshape(NB, BE, 1)
        m, ewb = _mlp_call(rt3, ug, ig, weights)
        np_part, dp_part = scatter(m, ewb, col_i[sl], np_part, dp_part)

    z = pl.pallas_call(
        _final_body,
        in_specs=[
            pl.BlockSpec((NC, IPAD, D_N), lambda: (0, 0, 0)),
            pl.BlockSpec((NC, IPAD, D_N), lambda: (0, 0, 0)),
            pl.BlockSpec((D_N, D_N), lambda: (0, 0)),
            pl.BlockSpec((1, D_N), lambda: (0, 0)),
        ],
        out_specs=pl.BlockSpec((IPAD, D_N), lambda: (0, 0)),
        out_shape=jax.ShapeDtypeStruct((IPAD, D_N), jnp.float32),
    )(np_part, dp_part, w_w.T, w_b.reshape(1, D_N))
    return z[:I_N]


# speedup vs baseline: 8.4997x; 1.0763x over previous
import functools

import jax
import jax.numpy as jnp
from jax import lax
from jax.experimental import pallas as pl
from jax.experimental.pallas import tpu as pltpu
from jax.experimental.pallas import tpu_sc as plsc

U_N, I_N, E_N, D_N, R_N = 8000, 2000, 320000, 128, 5

NC, NS = 2, 16
NW = NC * NS

NSLAB = 5
ES = E_N // NSLAB

KC = 80
GA = 5
GE = KC * GA

IPAD = 2048
IPT = IPAD // NS

BE = 1000
NB = ES // BE


def _largest_chunk(n):
    for k in range(128, 0, -8):
        if n % k == 0:
            return k
    raise ValueError(n)


def _mesh():
    return plsc.VectorSubcoreMesh(
        core_axis_name="c", subcore_axis_name="s",
        num_cores=NC, num_subcores=NS)


@functools.lru_cache(maxsize=None)
def _build_gather_stage(es):
    etw = es // NS
    ngrp = etw // GE

    def _gather_body(uf, itf, rowi, coli, ug, ig, idx, buf, gsem, wsem):
        cid = lax.axis_index("c")
        sid = lax.axis_index("s")
        base = sid * etw

        def run(table, idx_src, out):
            def grp(g, carry):
                b = lax.rem(g, 2)
                off = base + g * GE

                @pl.when(g >= 2)
                def _drain_write():
                    pltpu.make_async_copy(
                        buf.at[b], out.at[pl.ds(off, GE)], wsem).wait()

                for j in range(GA):
                    pltpu.sync_copy(idx_src.at[pl.ds(off + j * KC, KC)],
                                    idx.at[b, j])
                cps = [
                    pltpu.async_copy(
                        table.at[idx.at[b, j]],
                        buf.at[b].at[pl.ds(j * KC, KC)], gsem)
                    for j in range(GA)
                ]
                for c in cps:
                    c.wait()
                pltpu.async_copy(buf.at[b], out.at[pl.ds(off, GE)], wsem)
                return carry

            lax.fori_loop(0, ngrp, grp, 0)
            for b in range(2):
                pltpu.make_async_copy(
                    buf.at[b], out.at[pl.ds(base, GE)], wsem).wait()

        @pl.when(cid == 0)
        def _user():
            run(uf, rowi, ug)

        @pl.when(cid == 1)
        def _item():
            run(itf, coli, ig)

    return functools.partial(
        pl.kernel,
        out_type=[
            jax.ShapeDtypeStruct((es, D_N), jnp.float32),
            jax.ShapeDtypeStruct((es, D_N), jnp.float32),
        ],
        mesh=_mesh(),
        scratch_types=[
            pltpu.VMEM((2, GA, KC), jnp.int32),
            pltpu.VMEM((2, GE, D_N), jnp.float32),
            pltpu.SemaphoreType.DMA,
            pltpu.SemaphoreType.DMA,
        ],
    )(_gather_body)


def _mlp_body(rt_ref, u_ref, i_ref, w1u_ref, rf8_ref, w1r_ref, b1_ref,
              w2_ref, b2_ref, a1f_ref, a1i_ref, ab1_ref, a2_ref, ab2_ref,
              a3_ref, ab3_ref, m_ref, ew_ref):
    u = u_ref[...]
    iv = i_ref[...]
    r = rt_ref[0]
    oh = (r == lax.broadcasted_iota(jnp.int32, (BE, 8), 1)).astype(jnp.float32)
    rp = jnp.dot(rf8_ref[...], w1r_ref[...], preferred_element_type=jnp.float32)
    h = jnp.maximum(
        jnp.dot(u, w1u_ref[...], preferred_element_type=jnp.float32)
        + jnp.dot(oh, rp, preferred_element_type=jnp.float32)
        + b1_ref[...], 0.0)
    f = jnp.maximum(
        jnp.dot(h, w2_ref[...], preferred_element_type=jnp.float32)
        + b2_ref[...], 0.0)
    a = jnp.maximum(
        jnp.dot(f, a1f_ref[...], preferred_element_type=jnp.float32)
        + jnp.dot(iv, a1i_ref[...], preferred_element_type=jnp.float32)
        + ab1_ref[...], 0.0)
    a = jnp.maximum(
        jnp.dot(a, a2_ref[...], preferred_element_type=jnp.float32)
        + ab2_ref[...], 0.0)
    w8 = jnp.dot(a, a3_ref[...], preferred_element_type=jnp.float32) + ab3_ref[...]
    ew = jnp.exp(w8[:, 0:1])
    m_ref[...] = f * ew
    ew_ref[...] = jnp.broadcast_to(ew, (BE, D_N))


@functools.lru_cache(maxsize=None)
def _build_scatter_stage(es):
    epw = es // NW
    ks = _largest_chunk(epw)
    nch = epw // ks

    def _scatter_body(m, ew, coli, n_init, d_init, np_out, dp_out,
                      idxb, mbuf, ebuf, nbuf, lsem, accN, accD):
        cid = lax.axis_index("c")
        sid = lax.axis_index("s")
        wid = sid * NC + cid
        base = wid * epw

        def fire_loads(c, b):
            off = base + c * ks
            pltpu.async_copy(coli.at[pl.ds(off, ks)], idxb.at[b], lsem)
            pltpu.async_copy(m.at[pl.ds(off, ks)], mbuf.at[b], lsem)
            pltpu.async_copy(ew.at[pl.ds(off, ks)], ebuf.at[b], lsem)

        fire_loads(0, 0)
        pltpu.sync_copy(n_init.at[cid].at[pl.ds(sid * IPT, IPT)], nbuf)
        pltpu.sync_copy(nbuf, accN.at[pl.ds(sid * IPT, IPT)])
        pltpu.sync_copy(d_init.at[cid].at[pl.ds(sid * IPT, IPT)], nbuf)
        pltpu.sync_copy(nbuf, accD.at[pl.ds(sid * IPT, IPT)])
        plsc.subcore_barrier()

        def chunk(c, carry):
            b = lax.rem(c, 2)
            off = base + c * ks
            pltpu.make_async_copy(
                coli.at[pl.ds(off, ks)], idxb.at[b], lsem).wait()
            pltpu.make_async_copy(
                m.at[pl.ds(off, ks)], mbuf.at[b], lsem).wait()
            pltpu.make_async_copy(
                ew.at[pl.ds(off, ks)], ebuf.at[b], lsem).wait()

            @pl.when(c + 1 < nch)
            def _prefetch():
                fire_loads(c + 1, 1 - b)

            pltpu.sync_copy(mbuf.at[b], accN.at[idxb.at[b]], add=True)
            pltpu.sync_copy(ebuf.at[b], accD.at[idxb.at[b]], add=True)
            return carry

        lax.fori_loop(0, nch, chunk, 0)
        plsc.subcore_barrier()
        pltpu.sync_copy(accN.at[pl.ds(sid * IPT, IPT)], nbuf)
        pltpu.sync_copy(nbuf, np_out.at[cid].at[pl.ds(sid * IPT, IPT)])
        pltpu.sync_copy(accD.at[pl.ds(sid * IPT, IPT)], nbuf)
        pltpu.sync_copy(nbuf, dp_out.at[cid].at[pl.ds(sid * IPT, IPT)])

    return functools.partial(
        pl.kernel,
        out_type=[
            jax.ShapeDtypeStruct((NC, IPAD, D_N), jnp.float32),
            jax.ShapeDtypeStruct((NC, IPAD, D_N), jnp.float32),
        ],
        mesh=_mesh(),
        scratch_types=[
            pltpu.VMEM((2, ks), jnp.int32),
            pltpu.VMEM((2, ks, D_N), jnp.float32),
            pltpu.VMEM((2, ks, D_N), jnp.float32),
            pltpu.VMEM((IPT, D_N), jnp.float32),
            pltpu.SemaphoreType.DMA,
            pltpu.VMEM_SHARED((IPAD, D_N), jnp.float32),
            pltpu.VMEM_SHARED((IPAD, D_N), jnp.float32),
        ],
    )(_scatter_body)


def _final_body(np_ref, dp_ref, ww_ref, wb_ref, z_ref):
    n = np_ref[0] + np_ref[1]
    d = dp_ref[0, :, 0:1] + dp_ref[1, :, 0:1]
    d = jnp.where(d > 0, d, 1.0)
    z_ref[...] = (
        jnp.dot(n / d, ww_ref[...], preferred_element_type=jnp.float32)
        + wb_ref[...])


def _mlp_call(rt3, ug, ig, weights):
    full = lambda shp: pl.BlockSpec(shp, lambda b: tuple(0 for _ in shp))
    return pl.pallas_call(
        _mlp_body,
        grid=(NB,),
        in_specs=[
            pl.BlockSpec((1, BE, 1), lambda b: (b, 0, 0)),
            pl.BlockSpec((BE, D_N), lambda b: (b, 0)),
            pl.BlockSpec((BE, D_N), lambda b: (b, 0)),
            full((D_N, D_N)),
            full((8, D_N)),
            full((D_N, D_N)),
            full((1, D_N)),
            full((D_N, D_N)),
            full((1, D_N)),
            full((D_N, D_N)),
            full((D_N, D_N)),
            full((1, D_N)),
            full((D_N, D_N)),
            full((1, D_N)),
            full((D_N, 8)),
            full((1, 8)),
        ],
        out_specs=[
            pl.BlockSpec((BE, D_N), lambda b: (b, 0)),
            pl.BlockSpec((BE, D_N), lambda b: (b, 0)),
        ],
        out_shape=[
            jax.ShapeDtypeStruct((ES, D_N), jnp.float32),
            jax.ShapeDtypeStruct((ES, D_N), jnp.float32),
        ],
    )(rt3, ug, ig, *weights)


def kernel(user_feat, item_feat, rating_feat, row_idxs, col_idxs, rating,
           gu_w1, gu_b1, gu_w2, gu_b2,
           att_w1, att_b1, att_w2, att_b2, att_w3, att_b3,
           w_w, w_b):
    row_i = row_idxs.astype(jnp.int32)
    col_i = col_idxs.astype(jnp.int32)
    rat_i = rating.astype(jnp.int32)

    w1u = gu_w1[:, :D_N].T
    w1r = gu_w1[:, D_N:].T
    rf8 = jnp.zeros((8, D_N), jnp.float32).at[:R_N].set(rating_feat)
    w2t = gu_w2.T
    a1ft = att_w1[:, :D_N].T
    a1it = att_w1[:, D_N:].T
    a2t = att_w2.T
    a3p = jnp.zeros((D_N, 8), jnp.float32).at[:, 0].set(att_w3[0])
    b1_2 = gu_b1.reshape(1, D_N)
    b2_2 = gu_b2.reshape(1, D_N)
    ab1_2 = att_b1.reshape(1, D_N)
    ab2_2 = att_b2.reshape(1, D_N)
    ab3_2 = jnp.broadcast_to(att_b3.reshape(1, 1), (1, 8))
    weights = (w1u, rf8, w1r, b1_2, w2t, b2_2,
               a1ft, a1it, ab1_2, a2t, ab2_2, a3p, ab3_2)

    gather = _build_gather_stage(ES)
    scatter = _build_scatter_stage(ES)

    np_part = jnp.zeros((NC, IPAD, D_N), jnp.float32)
    dp_part = jnp.zeros((NC, IPAD, D_N), jnp.float32)
    for s in range(NSLAB):
        sl = slice(s * ES, (s + 1) * ES)
        ug, ig = gather(user_feat, item_feat, row_i[sl], col_i[sl])
        rt3 = rat_i[sl].reshape(NB, BE, 1)
        m, ewb = _mlp_call(rt3, ug, ig, weights)
        np_part, dp_part = scatter(m, ewb, col_i[sl], np_part, dp_part)

    z = pl.pallas_call(
        _final_body,
        in_specs=[
            pl.BlockSpec((NC, IPAD, D_N), lambda: (0, 0, 0)),
            pl.BlockSpec((NC, IPAD, D_N), lambda: (0, 0, 0)),
            pl.BlockSpec((D_N, D_N), lambda: (0, 0)),
            pl.BlockSpec((1, D_N), lambda: (0, 0)),
        ],
        out_specs=pl.BlockSpec((IPAD, D_N), lambda: (0, 0)),
        out_shape=jax.ShapeDtypeStruct((IPAD, D_N), jnp.float32),
    )(np_part, dp_part, w_w.T, w_b.reshape(1, D_N))
    return z[:I_N]
